# initial kernel scaffold (unmeasured)
import jax
import jax.numpy as jnp
from jax import lax
from jax.experimental import pallas as pl
from jax.experimental.pallas import tpu as pltpu

N_DEV = 4


def kernel(x, w_mat, scale_x, scale_w):
    m_total, k_per = x.shape
    _, n = w_mat.shape
    m_per = m_total // N_DEV

    scale = (scale_x.astype(jnp.float32) * scale_w.astype(jnp.float32)).reshape(1, 1)

    def body(scale_ref, x_ref, w_ref, out_ref, comm_ref, send_sems, recv_sems):
        my = lax.axis_index("i")
        left = lax.rem(my + N_DEV - 1, N_DEV)
        right = lax.rem(my + 1, N_DEV)

        barrier = pltpu.get_barrier_semaphore()
        for nbr in (left, right):
            pl.semaphore_signal(
                barrier, inc=1,
                device_id=(nbr,), device_id_type=pl.DeviceIdType.MESH,
            )
        pl.semaphore_wait(barrier, 2)

        w_bf = w_ref[...].astype(jnp.bfloat16)

        def partial(c):
            xa = x_ref[pl.ds(c * m_per, m_per), :].astype(jnp.bfloat16)
            return jnp.dot(xa, w_bf, preferred_element_type=jnp.float32)

        comm_ref[0, :, :] = partial(lax.rem(my + N_DEV - 1, N_DEV)).astype(
            jnp.bfloat16
        )

        for h in range(N_DEV - 1):
            rdma = pltpu.make_async_remote_copy(
                src_ref=comm_ref.at[h],
                dst_ref=comm_ref.at[h + 1],
                send_sem=send_sems.at[h],
                recv_sem=recv_sems.at[h],
                device_id=(right,),
                device_id_type=pl.DeviceIdType.MESH,
            )
            rdma.start()
            rdma.wait()

            p = partial(lax.rem(my + N_DEV - 2 - h, N_DEV))
            if h < N_DEV - 2:
                comm_ref[h + 1, :, :] = (
                    comm_ref[h + 1, :, :].astype(jnp.float32) + p
                ).astype(jnp.bfloat16)
            else:
                acc = comm_ref[h + 1, :, :].astype(jnp.float32) + p
                y = acc * scale_ref[0, 0]
                out_ref[...] = y * jax.nn.sigmoid(y)

    return pl.pallas_call(
        body,
        out_shape=jax.ShapeDtypeStruct((m_per, n), jnp.float32),
        in_specs=[
            pl.BlockSpec(memory_space=pltpu.SMEM),
            pl.BlockSpec(memory_space=pltpu.VMEM),
            pl.BlockSpec(memory_space=pltpu.VMEM),
        ],
        out_specs=pl.BlockSpec(memory_space=pltpu.VMEM),
        scratch_shapes=[
            pltpu.VMEM((N_DEV, m_per, n), jnp.bfloat16),
            pltpu.SemaphoreType.DMA((N_DEV - 1,)),
            pltpu.SemaphoreType.DMA((N_DEV - 1,)),
        ],
        compiler_params=pltpu.CompilerParams(collective_id=0),
    )(scale, x, w_mat)


# baseline (device time: 179777 ns/iter reference)
import jax
import jax.numpy as jnp
from jax import lax
from jax.experimental import pallas as pl
from jax.experimental.pallas import tpu as pltpu

N_DEV = 4


def kernel(x, w_mat, scale_x, scale_w):
    m_total, k_per = x.shape
    _, n = w_mat.shape
    m_per = m_total // N_DEV

    scale = (scale_x.astype(jnp.float32) * scale_w.astype(jnp.float32)).reshape(1, 1)

    def body(scale_ref, x_ref, w_ref, out_ref, comm_ref, send_sems, recv_sems):
        my = lax.axis_index("i")
        left = lax.rem(my + N_DEV - 1, N_DEV)
        right = lax.rem(my + 1, N_DEV)

        barrier = pltpu.get_barrier_semaphore()
        for nbr in (left, right):
            pl.semaphore_signal(
                barrier, inc=1,
                device_id=(nbr,), device_id_type=pl.DeviceIdType.MESH,
            )
        pl.semaphore_wait(barrier, 2)

        w_bf = w_ref[...].astype(jnp.bfloat16)

        def partial(c):
            xa = x_ref[pl.ds(c * m_per, m_per), :].astype(jnp.bfloat16)
            return jnp.dot(xa, w_bf, preferred_element_type=jnp.float32)

        comm_ref[0, :, :] = partial(lax.rem(my + N_DEV - 1, N_DEV)).astype(
            jnp.bfloat16
        )

        for h in range(N_DEV - 1):
            rdma = pltpu.make_async_remote_copy(
                src_ref=comm_ref.at[h],
                dst_ref=comm_ref.at[h + 1],
                send_sem=send_sems.at[h],
                recv_sem=recv_sems.at[h],
                device_id=(right,),
                device_id_type=pl.DeviceIdType.MESH,
            )
            rdma.start()
            rdma.wait()

            p = partial(lax.rem(my + N_DEV - 2 - h, N_DEV))
            if h < N_DEV - 2:
                comm_ref[h + 1, :, :] = (
                    comm_ref[h + 1, :, :].astype(jnp.float32) + p
                ).astype(jnp.bfloat16)
            else:
                acc = comm_ref[h + 1, :, :].astype(jnp.float32) + p
                y = acc * scale_ref[0, 0]
                out_ref[...] = y * jax.nn.sigmoid(y)

    return pl.pallas_call(
        body,
        out_shape=jax.ShapeDtypeStruct((m_per, n), jnp.float32),
        in_specs=[
            pl.BlockSpec(memory_space=pltpu.SMEM),
            pl.BlockSpec(memory_space=pltpu.VMEM),
            pl.BlockSpec(memory_space=pltpu.VMEM),
        ],
        out_specs=pl.BlockSpec(memory_space=pltpu.VMEM),
        scratch_shapes=[
            pltpu.VMEM((N_DEV, m_per, n), jnp.bfloat16),
            pltpu.SemaphoreType.DMA((N_DEV - 1,)),
            pltpu.SemaphoreType.DMA((N_DEV - 1,)),
        ],
        compiler_params=pltpu.CompilerParams(
            collective_id=0,
            vmem_limit_bytes=100 * 1024 * 1024,
        ),
    )(scale, x, w_mat)


# device time: 103050 ns/iter; 1.7446x vs baseline; 1.7446x over previous
import jax
import jax.numpy as jnp
from jax import lax
from jax.experimental import pallas as pl
from jax.experimental.pallas import tpu as pltpu

N_DEV = 4


def kernel(x, w_mat, scale_x, scale_w):
    m_total, k_per = x.shape
    _, n = w_mat.shape
    m_per = m_total // N_DEV
    nh = n // 2

    scale = (scale_x.astype(jnp.float32) * scale_w.astype(jnp.float32)).reshape(1, 1)

    def body(scale_ref, x_ref, w_ref, out_ref,
             cw_buf, ccw_buf, cw_send, cw_recv, ccw_send, ccw_recv):
        my = lax.axis_index("i")
        left = lax.rem(my + N_DEV - 1, N_DEV)
        right = lax.rem(my + 1, N_DEV)

        barrier = pltpu.get_barrier_semaphore()
        for nbr in (left, right):
            pl.semaphore_signal(
                barrier, inc=1,
                device_id=(nbr,), device_id_type=pl.DeviceIdType.MESH,
            )
        pl.semaphore_wait(barrier, 2)

        w_bf = w_ref[...].astype(jnp.bfloat16)

        def p_left(c):
            xa = x_ref[pl.ds(c * m_per, m_per), :].astype(jnp.bfloat16)
            return jnp.dot(xa, w_bf[:, :nh], preferred_element_type=jnp.float32)

        def p_right(c):
            xa = x_ref[pl.ds(c * m_per, m_per), :].astype(jnp.bfloat16)
            return jnp.dot(xa, w_bf[:, nh:], preferred_element_type=jnp.float32)

        cw_buf[0, :, :] = p_left(lax.rem(my + N_DEV - 1, N_DEV)).astype(jnp.bfloat16)
        ccw_buf[0, :, :] = p_right(lax.rem(my + 1, N_DEV)).astype(jnp.bfloat16)

        for h in range(N_DEV - 1):
            r_cw = pltpu.make_async_remote_copy(
                src_ref=cw_buf.at[h], dst_ref=cw_buf.at[h + 1],
                send_sem=cw_send.at[h], recv_sem=cw_recv.at[h],
                device_id=(right,), device_id_type=pl.DeviceIdType.MESH,
            )
            r_ccw = pltpu.make_async_remote_copy(
                src_ref=ccw_buf.at[h], dst_ref=ccw_buf.at[h + 1],
                send_sem=ccw_send.at[h], recv_sem=ccw_recv.at[h],
                device_id=(left,), device_id_type=pl.DeviceIdType.MESH,
            )
            r_cw.start()
            r_ccw.start()

            p_cw = p_left(lax.rem(my + 2 - h + N_DEV, N_DEV))
            p_ccw = p_right(lax.rem(my + 2 + h, N_DEV))

            if h < N_DEV - 2:
                r_cw.wait()
                cw_buf[h + 1, :, :] = (
                    cw_buf[h + 1, :, :].astype(jnp.float32) + p_cw
                ).astype(jnp.bfloat16)
                r_ccw.wait()
                ccw_buf[h + 1, :, :] = (
                    ccw_buf[h + 1, :, :].astype(jnp.float32) + p_ccw
                ).astype(jnp.bfloat16)
            else:
                s = scale_ref[0, 0]
                r_cw.wait()
                y = (cw_buf[h + 1, :, :].astype(jnp.float32) + p_cw) * s
                out_ref[:, :nh] = y * jax.nn.sigmoid(y)
                r_ccw.wait()
                y = (ccw_buf[h + 1, :, :].astype(jnp.float32) + p_ccw) * s
                out_ref[:, nh:] = y * jax.nn.sigmoid(y)

    return pl.pallas_call(
        body,
        out_shape=jax.ShapeDtypeStruct((m_per, n), jnp.float32),
        in_specs=[
            pl.BlockSpec(memory_space=pltpu.SMEM),
            pl.BlockSpec(memory_space=pltpu.VMEM),
            pl.BlockSpec(memory_space=pltpu.VMEM),
        ],
        out_specs=pl.BlockSpec(memory_space=pltpu.VMEM),
        scratch_shapes=[
            pltpu.VMEM((N_DEV, m_per, nh), jnp.bfloat16),
            pltpu.VMEM((N_DEV, m_per, nh), jnp.bfloat16),
            pltpu.SemaphoreType.DMA((N_DEV - 1,)),
            pltpu.SemaphoreType.DMA((N_DEV - 1,)),
            pltpu.SemaphoreType.DMA((N_DEV - 1,)),
            pltpu.SemaphoreType.DMA((N_DEV - 1,)),
        ],
        compiler_params=pltpu.CompilerParams(
            collective_id=0,
            vmem_limit_bytes=100 * 1024 * 1024,
        ),
    )(scale, x, w_mat)


# device time: 93897 ns/iter; 1.9146x vs baseline; 1.0975x over previous
import jax
import jax.numpy as jnp
from jax import lax
from jax.experimental import pallas as pl
from jax.experimental.pallas import tpu as pltpu

N_DEV = 4
S = 2


def kernel(x, w_mat, scale_x, scale_w):
    m_total, k_per = x.shape
    _, n = w_mat.shape
    m_per = m_total // N_DEV
    m_sub = m_per // S
    nh = n // 2

    scale = (scale_x.astype(jnp.float32) * scale_w.astype(jnp.float32)).reshape(1, 1)

    def body(scale_ref, x_ref, w_ref, out_ref,
             cw_buf, ccw_buf, cw_send, cw_recv, ccw_send, ccw_recv):
        my = lax.axis_index("i")
        left = lax.rem(my + N_DEV - 1, N_DEV)
        right = lax.rem(my + 1, N_DEV)

        barrier = pltpu.get_barrier_semaphore()
        for nbr in (left, right):
            pl.semaphore_signal(
                barrier, inc=1,
                device_id=(nbr,), device_id_type=pl.DeviceIdType.MESH,
            )
        pl.semaphore_wait(barrier, 2)

        sc = scale_ref[0, 0]
        w_bf = w_ref[...].astype(jnp.bfloat16)
        w_halves = (w_bf[:, :nh], w_bf[:, nh:])

        def psub(c, s, half):
            xa = x_ref[pl.ds(c * m_per + s * m_sub, m_sub), :].astype(jnp.bfloat16)
            return jnp.dot(xa, w_halves[half], preferred_element_type=jnp.float32)

        def mk(buf, send_sems, recv_sems, tgt, h, s):
            return pltpu.make_async_remote_copy(
                src_ref=buf.at[h, s],
                dst_ref=buf.at[h + 1, s],
                send_sem=send_sems.at[h, s],
                recv_sem=recv_sems.at[h, s],
                device_id=(tgt,),
                device_id_type=pl.DeviceIdType.MESH,
            )

        descs = {}

        for s in range(S):
            cw_buf[0, s] = psub(lax.rem(my + N_DEV - 1, N_DEV), s, 0).astype(
                jnp.bfloat16
            )
            d = mk(cw_buf, cw_send, cw_recv, right, 0, s)
            d.start()
            descs["cw", 0, s] = d
            ccw_buf[0, s] = psub(lax.rem(my + 1, N_DEV), s, 1).astype(jnp.bfloat16)
            d = mk(ccw_buf, ccw_send, ccw_recv, left, 0, s)
            d.start()
            descs["ccw", 0, s] = d

        for h in range(N_DEV - 1):
            c_cw = lax.rem(my + 2 - h + N_DEV, N_DEV)
            c_ccw = lax.rem(my + 2 + h, N_DEV)
            for s in range(S):
                p_cw = psub(c_cw, s, 0)
                p_ccw = psub(c_ccw, s, 1)

                descs["cw", h, s].wait_recv()
                acc = cw_buf[h + 1, s].astype(jnp.float32) + p_cw
                if h < N_DEV - 2:
                    cw_buf[h + 1, s] = acc.astype(jnp.bfloat16)
                    d = mk(cw_buf, cw_send, cw_recv, right, h + 1, s)
                    d.start()
                    descs["cw", h + 1, s] = d
                else:
                    y = acc * sc
                    out_ref[pl.ds(s * m_sub, m_sub), pl.ds(0, nh)] = (
                        y * jax.nn.sigmoid(y)
                    )

                descs["ccw", h, s].wait_recv()
                acc = ccw_buf[h + 1, s].astype(jnp.float32) + p_ccw
                if h < N_DEV - 2:
                    ccw_buf[h + 1, s] = acc.astype(jnp.bfloat16)
                    d = mk(ccw_buf, ccw_send, ccw_recv, left, h + 1, s)
                    d.start()
                    descs["ccw", h + 1, s] = d
                else:
                    y = acc * sc
                    out_ref[pl.ds(s * m_sub, m_sub), pl.ds(nh, nh)] = (
                        y * jax.nn.sigmoid(y)
                    )

        for d in descs.values():
            d.wait_send()

    return pl.pallas_call(
        body,
        out_shape=jax.ShapeDtypeStruct((m_per, n), jnp.float32),
        in_specs=[
            pl.BlockSpec(memory_space=pltpu.SMEM),
            pl.BlockSpec(memory_space=pltpu.VMEM),
            pl.BlockSpec(memory_space=pltpu.VMEM),
        ],
        out_specs=pl.BlockSpec(memory_space=pltpu.VMEM),
        scratch_shapes=[
            pltpu.VMEM((N_DEV, S, m_sub, nh), jnp.bfloat16),
            pltpu.VMEM((N_DEV, S, m_sub, nh), jnp.bfloat16),
            pltpu.SemaphoreType.DMA((N_DEV - 1, S)),
            pltpu.SemaphoreType.DMA((N_DEV - 1, S)),
            pltpu.SemaphoreType.DMA((N_DEV - 1, S)),
            pltpu.SemaphoreType.DMA((N_DEV - 1, S)),
        ],
        compiler_params=pltpu.CompilerParams(
            collective_id=0,
            vmem_limit_bytes=100 * 1024 * 1024,
        ),
    )(scale, x, w_mat)


# device time: 92603 ns/iter; 1.9414x vs baseline; 1.0140x over previous
import jax
import jax.numpy as jnp
from jax import lax
from jax.experimental import pallas as pl
from jax.experimental.pallas import tpu as pltpu

N_DEV = 4
S = 4


def kernel(x, w_mat, scale_x, scale_w):
    m_total, k_per = x.shape
    _, n = w_mat.shape
    m_per = m_total // N_DEV
    m_sub = m_per // S
    nh = n // 2

    scale = (scale_x.astype(jnp.float32) * scale_w.astype(jnp.float32)).reshape(1, 1)

    def body(scale_ref, x_ref, w_ref, out_ref,
             cw_buf, ccw_buf, cw_send, cw_recv, ccw_send, ccw_recv):
        my = lax.axis_index("i")
        left = lax.rem(my + N_DEV - 1, N_DEV)
        right = lax.rem(my + 1, N_DEV)

        barrier = pltpu.get_barrier_semaphore()
        for nbr in (left, right):
            pl.semaphore_signal(
                barrier, inc=1,
                device_id=(nbr,), device_id_type=pl.DeviceIdType.MESH,
            )
        pl.semaphore_wait(barrier, 2)

        sc = scale_ref[0, 0]
        w_bf = w_ref[...].astype(jnp.bfloat16)
        w_halves = (w_bf[:, :nh], w_bf[:, nh:])

        def psub(c, s, half):
            xa = x_ref[pl.ds(c * m_per + s * m_sub, m_sub), :].astype(jnp.bfloat16)
            return jnp.dot(xa, w_halves[half], preferred_element_type=jnp.float32)

        def mk(buf, send_sems, recv_sems, tgt, h, s):
            return pltpu.make_async_remote_copy(
                src_ref=buf.at[h, s],
                dst_ref=buf.at[h + 1, s],
                send_sem=send_sems.at[h, s],
                recv_sem=recv_sems.at[h, s],
                device_id=(tgt,),
                device_id_type=pl.DeviceIdType.MESH,
            )

        descs = {}

        for s in range(S):
            cw_buf[0, s] = psub(lax.rem(my + N_DEV - 1, N_DEV), s, 0).astype(
                jnp.bfloat16
            )
            d = mk(cw_buf, cw_send, cw_recv, right, 0, s)
            d.start()
            descs["cw", 0, s] = d
            ccw_buf[0, s] = psub(lax.rem(my + 1, N_DEV), s, 1).astype(jnp.bfloat16)
            d = mk(ccw_buf, ccw_send, ccw_recv, left, 0, s)
            d.start()
            descs["ccw", 0, s] = d

        for h in range(N_DEV - 1):
            c_cw = lax.rem(my + 2 - h + N_DEV, N_DEV)
            c_ccw = lax.rem(my + 2 + h, N_DEV)
            for s in range(S):
                p_cw = psub(c_cw, s, 0)
                p_ccw = psub(c_ccw, s, 1)

                descs["cw", h, s].wait_recv()
                acc = cw_buf[h + 1, s].astype(jnp.float32) + p_cw
                if h < N_DEV - 2:
                    cw_buf[h + 1, s] = acc.astype(jnp.bfloat16)
                    d = mk(cw_buf, cw_send, cw_recv, right, h + 1, s)
                    d.start()
                    descs["cw", h + 1, s] = d
                else:
                    y = acc * sc
                    out_ref[pl.ds(s * m_sub, m_sub), pl.ds(0, nh)] = (
                        y * jax.nn.sigmoid(y)
                    )

                descs["ccw", h, s].wait_recv()
                acc = ccw_buf[h + 1, s].astype(jnp.float32) + p_ccw
                if h < N_DEV - 2:
                    ccw_buf[h + 1, s] = acc.astype(jnp.bfloat16)
                    d = mk(ccw_buf, ccw_send, ccw_recv, left, h + 1, s)
                    d.start()
                    descs["ccw", h + 1, s] = d
                else:
                    y = acc * sc
                    out_ref[pl.ds(s * m_sub, m_sub), pl.ds(nh, nh)] = (
                        y * jax.nn.sigmoid(y)
                    )

        for d in descs.values():
            d.wait_send()

    return pl.pallas_call(
        body,
        out_shape=jax.ShapeDtypeStruct((m_per, n), jnp.float32),
        in_specs=[
            pl.BlockSpec(memory_space=pltpu.SMEM),
            pl.BlockSpec(memory_space=pltpu.VMEM),
            pl.BlockSpec(memory_space=pltpu.VMEM),
        ],
        out_specs=pl.BlockSpec(memory_space=pltpu.VMEM),
        scratch_shapes=[
            pltpu.VMEM((N_DEV, S, m_sub, nh), jnp.bfloat16),
            pltpu.VMEM((N_DEV, S, m_sub, nh), jnp.bfloat16),
            pltpu.SemaphoreType.DMA((N_DEV - 1, S)),
            pltpu.SemaphoreType.DMA((N_DEV - 1, S)),
            pltpu.SemaphoreType.DMA((N_DEV - 1, S)),
            pltpu.SemaphoreType.DMA((N_DEV - 1, S)),
        ],
        compiler_params=pltpu.CompilerParams(
            collective_id=0,
            vmem_limit_bytes=100 * 1024 * 1024,
        ),
    )(scale, x, w_mat)


# device time: 86601 ns/iter; 2.0759x vs baseline; 1.0693x over previous
import jax
import jax.numpy as jnp
from jax import lax
from jax.experimental import pallas as pl
from jax.experimental.pallas import tpu as pltpu

N_DEV = 4
E4M3 = jnp.float8_e4m3fn
E5M2 = jnp.float8_e5m2


def kernel(x, w_mat, scale_x, scale_w):
    m_total, k_per = x.shape
    _, n = w_mat.shape
    m_per = m_total // N_DEV
    nh = n // 2

    scale = (scale_x.astype(jnp.float32) * scale_w.astype(jnp.float32)).reshape(1, 1)

    def body(scale_ref, x_ref, w_ref, out_ref,
             w8, x8, wcw, wccw, xrecv,
             send_sems, wcw_sems, wccw_sems, xrecv_sems, dummy_sem):
        my = lax.axis_index("i")
        left = lax.rem(my + N_DEV - 1, N_DEV)
        right = lax.rem(my + 1, N_DEV)
        diag = lax.rem(my + 2, N_DEV)

        barrier = pltpu.get_barrier_semaphore()
        for nbr in (left, right, diag):
            pl.semaphore_signal(
                barrier, inc=1,
                device_id=(nbr,), device_id_type=pl.DeviceIdType.MESH,
            )
        pl.semaphore_wait(barrier, 3)

        sc = scale_ref[0, 0]
        sends = []

        def send(src, dst, sem_i, tgt):
            d = pltpu.make_async_remote_copy(
                src_ref=src, dst_ref=dst,
                send_sem=send_sems.at[sem_i], recv_sem=dummy_sem.at[0],
                device_id=(tgt,), device_id_type=pl.DeviceIdType.MESH,
            )
            return d

        def start(src, dst, recv_sem, sem_i, tgt):
            d = pltpu.make_async_remote_copy(
                src_ref=src, dst_ref=dst,
                send_sem=send_sems.at[sem_i], recv_sem=recv_sem,
                device_id=(tgt,), device_id_type=pl.DeviceIdType.MESH,
            )
            d.start()
            sends.append(d)

        def wait_recv(buf_slot, recv_sem, src_dev):
            d = pltpu.make_async_remote_copy(
                src_ref=buf_slot, dst_ref=buf_slot,
                send_sem=dummy_sem.at[0], recv_sem=recv_sem,
                device_id=(src_dev,), device_id_type=pl.DeviceIdType.MESH,
            )
            d.wait_recv()

        w8[...] = w_ref[...].astype(E5M2)
        start(w8.at[:, pl.ds(0, nh)], wcw.at[0], wcw_sems.at[0], 0, right)
        start(w8.at[:, pl.ds(nh, nh)], wccw.at[0], wccw_sems.at[0], 1, left)

        x8[0] = x_ref[pl.ds(right * m_per, m_per), :].astype(E4M3)
        start(x8.at[0], xrecv.at[0], xrecv_sems.at[0], 2, right)
        x8[1] = x_ref[pl.ds(left * m_per, m_per), :].astype(E4M3)
        start(x8.at[1], xrecv.at[1], xrecv_sems.at[1], 3, left)
        x8[2] = x_ref[pl.ds(diag * m_per, m_per), :].astype(E4M3)
        start(x8.at[2], xrecv.at[2], xrecv_sems.at[2], 4, diag)

        w_bf = w_ref[...].astype(jnp.bfloat16)
        xa = x_ref[pl.ds(my * m_per, m_per), :].astype(jnp.bfloat16)
        out_ref[...] = jnp.dot(xa, w_bf, preferred_element_type=jnp.float32)

        def accum(cols0, xblk, wblk):
            d = jnp.dot(
                xblk.astype(jnp.bfloat16), wblk.astype(jnp.bfloat16),
                preferred_element_type=jnp.float32,
            )
            out_ref[:, pl.ds(cols0, nh)] = out_ref[:, pl.ds(cols0, nh)] + d

        wait_recv(wcw.at[0], wcw_sems.at[0], left)
        start(wcw.at[0], wcw.at[1], wcw_sems.at[1], 5, right)
        wait_recv(wccw.at[0], wccw_sems.at[0], right)
        start(wccw.at[0], wccw.at[1], wccw_sems.at[1], 6, left)

        wait_recv(xrecv.at[0], xrecv_sems.at[0], left)
        accum(0, xrecv[0], wcw[0])
        wait_recv(xrecv.at[1], xrecv_sems.at[1], right)
        accum(nh, xrecv[1], wccw[0])

        wait_recv(wcw.at[1], wcw_sems.at[1], left)
        start(wcw.at[1], wcw.at[2], wcw_sems.at[2], 7, right)
        wait_recv(wccw.at[1], wccw_sems.at[1], right)
        start(wccw.at[1], wccw.at[2], wccw_sems.at[2], 8, left)

        wait_recv(xrecv.at[2], xrecv_sems.at[2], diag)
        accum(0, xrecv[2], wcw[1])
        accum(nh, xrecv[2], wccw[1])

        def epilogue(cols0):
            y = out_ref[:, pl.ds(cols0, nh)] * sc
            out_ref[:, pl.ds(cols0, nh)] = y * jax.nn.sigmoid(y)

        wait_recv(wcw.at[2], wcw_sems.at[2], left)
        accum(0, xrecv[1], wcw[2])
        epilogue(0)
        wait_recv(wccw.at[2], wccw_sems.at[2], right)
        accum(nh, xrecv[0], wccw[2])
        epilogue(nh)

        for d in sends:
            d.wait_send()

    return pl.pallas_call(
        body,
        out_shape=jax.ShapeDtypeStruct((m_per, n), jnp.float32),
        in_specs=[
            pl.BlockSpec(memory_space=pltpu.SMEM),
            pl.BlockSpec(memory_space=pltpu.VMEM),
            pl.BlockSpec(memory_space=pltpu.VMEM),
        ],
        out_specs=pl.BlockSpec(memory_space=pltpu.VMEM),
        scratch_shapes=[
            pltpu.VMEM((k_per, n), E5M2),
            pltpu.VMEM((3, m_per, k_per), E4M3),
            pltpu.VMEM((3, k_per, nh), E5M2),
            pltpu.VMEM((3, k_per, nh), E5M2),
            pltpu.VMEM((3, m_per, k_per), E4M3),
            pltpu.SemaphoreType.DMA((9,)),
            pltpu.SemaphoreType.DMA((3,)),
            pltpu.SemaphoreType.DMA((3,)),
            pltpu.SemaphoreType.DMA((3,)),
            pltpu.SemaphoreType.DMA((1,)),
        ],
        compiler_params=pltpu.CompilerParams(
            collective_id=0,
            vmem_limit_bytes=100 * 1024 * 1024,
        ),
    )(scale, x, w_mat)
